# Initial kernel scaffold; baseline (speedup 1.0000x reference)
import jax
import jax.numpy as jnp
from jax.experimental import pallas as pl


def kernel(x, edge_index, edge_type, weight, comp, bias):
    raise NotImplementedError("write your pallas kernel here")


# speedup vs baseline: 20.6379x; 20.6379x over previous
import jax
import jax.numpy as jnp
from jax import lax
from jax.experimental import pallas as pl
from jax.experimental.pallas import tpu as pltpu
from jax.experimental.pallas import tpu_sc as plsc

N_NODES = 10000
NUM_REL = 8
D = 128
DH = D // 2

NC = 2
NS = 16
E_TOTAL = 320000
EPT = E_TOTAL // NS
K = 80
C = EPT // K
N_PAD = 10240
RPT = N_PAD // NS
ZC = 128


def _transform_body(x_ref, w_ref, comp_ref, h_ref):
    r = pl.program_id(1)
    rows = lax.broadcasted_iota(jnp.int32, (NUM_REL, NUM_REL), 0)
    scale = jnp.sum(jnp.where(rows == r, comp_ref[...], 0.0))
    h_ref[...] = jnp.dot(x_ref[...], w_ref[0, 0],
                         preferred_element_type=jnp.float32) * scale


def _transform(x, weight_split, comp):
    nb = 10
    bn = N_NODES // nb
    return pl.pallas_call(
        _transform_body,
        grid=(nb, NUM_REL, NC),
        in_specs=[
            pl.BlockSpec((bn, D), lambda i, r, c: (i, 0)),
            pl.BlockSpec((1, 1, D, DH), lambda i, r, c: (r, c, 0, 0)),
            pl.BlockSpec((NUM_REL, NUM_REL), lambda i, r, c: (0, 0)),
        ],
        out_specs=pl.BlockSpec((bn, DH),
                               lambda i, r, c: (c * NUM_REL * nb + r * nb + i, 0)),
        out_shape=jax.ShapeDtypeStruct((NC * NUM_REL * N_NODES, DH), jnp.float32),
    )(x, weight_split, comp)


def _sc_body(h_hbm, et_hbm, col_hbm, row_hbm, out_hbm,
             a_v, b_v, rows_v, zb_v, acc, sem):
    cid = lax.axis_index("c")
    sid = lax.axis_index("s")

    pltpu.sync_copy(et_hbm.at[sid], a_v)
    pltpu.sync_copy(col_hbm.at[sid], b_v)

    goff = cid * NUM_REL * N_NODES
    def g_body(j, carry):
        for i in range(K // 16):
            s = pl.ds(i * 16, 16)
            a_v[j, s] = a_v[j, s] * N_NODES + b_v[j, s] + goff
        return carry
    lax.fori_loop(0, C, g_body, 0)

    pltpu.sync_copy(row_hbm.at[sid], b_v)

    def zb_body(j, carry):
        for i in range(DH // 16):
            zb_v[j, pl.ds(i * 16, 16)] = jnp.zeros((16,), jnp.float32)
        return carry
    lax.fori_loop(0, ZC, zb_body, 0)
    base = sid * RPT
    for z in range(RPT // ZC):
        pltpu.sync_copy(zb_v, acc.at[pl.ds(base + z * ZC, ZC)])

    plsc.subcore_barrier()

    def m_body(j, carry):
        pltpu.async_copy(h_hbm.at[a_v.at[j]], rows_v, sem).wait()
        pltpu.sync_copy(rows_v, acc.at[b_v.at[j]], add=True)
        return carry
    lax.fori_loop(0, C, m_body, 0)

    plsc.subcore_barrier()

    for z in range(RPT // ZC):
        s = pl.ds(base + z * ZC, ZC)
        pltpu.sync_copy(acc.at[s], out_hbm.at[cid, s])


def _sc_scatter(h, et, col, row):
    mesh = plsc.VectorSubcoreMesh(core_axis_name="c", subcore_axis_name="s",
                                  num_cores=NC, num_subcores=NS)
    f = pl.kernel(
        _sc_body,
        out_type=jax.ShapeDtypeStruct((NC, N_PAD, DH), jnp.float32),
        mesh=mesh,
        scratch_types=[
            pltpu.VMEM((C, K), jnp.int32),
            pltpu.VMEM((C, K), jnp.int32),
            pltpu.VMEM((K, DH), jnp.float32),
            pltpu.VMEM((ZC, DH), jnp.float32),
            pltpu.VMEM_SHARED((N_PAD, DH), jnp.float32),
            pltpu.SemaphoreType.DMA,
        ],
        compiler_params=pltpu.CompilerParams(use_tc_tiling_on_sc=False),
    )
    return f(h, et, col, row)


def _combine_body(p_ref, b_ref, o_ref):
    full = jnp.concatenate([p_ref[0], p_ref[1]], axis=1)
    o_ref[...] = jnp.maximum(full + b_ref[...], 0.0)


def _combine(partial, bias2d):
    nb = 10
    bn = N_NODES // nb
    return pl.pallas_call(
        _combine_body,
        grid=(nb,),
        in_specs=[
            pl.BlockSpec((NC, bn, DH), lambda i: (0, i, 0)),
            pl.BlockSpec((1, D), lambda i: (0, 0)),
        ],
        out_specs=pl.BlockSpec((bn, D), lambda i: (i, 0)),
        out_shape=jax.ShapeDtypeStruct((N_NODES, D), jnp.float32),
    )(partial, bias2d)


def kernel(x, edge_index, edge_type, weight, comp, bias):
    row = edge_index[0].reshape(NS, C, K)
    col = edge_index[1].reshape(NS, C, K)
    et = edge_type.reshape(NS, C, K)
    w_split = weight.reshape(NUM_REL, D, NC, DH).transpose(0, 2, 1, 3)
    h = _transform(x, w_split, comp)
    partial = _sc_scatter(h, et, col, row)
    return _combine(partial, bias.reshape(1, D))
